# Initial kernel scaffold; baseline (speedup 1.0000x reference)
import jax
import jax.numpy as jnp
from jax.experimental import pallas as pl


def kernel(feat_ids, edge_index_entail, edge_index_pathway, embed_table, W_in, b_in, ln_in_s, ln_in_b, W_self, W_neigh, b_conv, ln_c_s, ln_c_b, W_ro, b_ro):
    raise NotImplementedError("write your pallas kernel here")


# speedup vs baseline: 5.4859x; 5.4859x over previous
import functools

import jax
import jax.numpy as jnp
from jax import lax
from jax.experimental import pallas as pl
from jax.experimental.pallas import tpu as pltpu
from jax.experimental.pallas import tpu_sc as plsc

N = 10000
E = 160000
HID = 128
NPAD = 10240
OWN = 640
CHUNK = 125
NCHUNK = 80
EROWS = 320
ECHUNK = 80
ENCHUNK = 4

_F32 = jnp.float32


@functools.lru_cache(maxsize=None)
def _sc_mesh():
    return plsc.VectorSubcoreMesh(
        core_axis_name="c", subcore_axis_name="s", num_cores=2, num_subcores=16
    )


def _ln(x, s, b):
    m = jnp.mean(x, axis=-1, keepdims=True)
    v = jnp.mean((x - m) ** 2, axis=-1, keepdims=True)
    return (x - m) / jnp.sqrt(v + 1e-5) * s + b


def _sc_embed(fid4, table):
    @functools.partial(
        pl.kernel,
        out_type=jax.ShapeDtypeStruct((2, 16, ENCHUNK, ECHUNK, HID), _F32),
        mesh=_sc_mesh(),
        scratch_types=[
            pltpu.VMEM((ENCHUNK, ECHUNK), jnp.int32),
            pltpu.VMEM((ECHUNK, HID), _F32),
            pltpu.SemaphoreType.DMA,
        ],
    )
    def k(fid_h, table_h, h0_out, fidv, ebuf, sem):
        c = lax.axis_index("c")
        s = lax.axis_index("s")
        pltpu.sync_copy(fid_h.at[c, s], fidv)

        def ebody(j, carry):
            pltpu.async_copy(table_h.at[fidv.at[j]], ebuf, sem).wait()
            pltpu.sync_copy(ebuf, h0_out.at[c, s, j])
            return carry

        lax.fori_loop(0, ENCHUNK, ebody, 0)

    return k(fid4, table)


def _sc_deg(dst3, zeros128, ones128):
    @functools.partial(
        pl.kernel,
        out_type=jax.ShapeDtypeStruct((2, 16, 5, 128, HID), _F32),
        mesh=_sc_mesh(),
        scratch_types=[
            pltpu.VMEM((NCHUNK, CHUNK), jnp.int32),
            pltpu.VMEM((128, HID), _F32),
            pltpu.VMEM_SHARED((NPAD, HID), _F32),
        ],
    )
    def k(dst_h, z_h, ones_h, deg_out, dstv, buf, acc):
        c = lax.axis_index("c")
        s = lax.axis_index("s")
        pltpu.sync_copy(dst_h.at[c, s], dstv)
        pltpu.sync_copy(z_h, buf)
        for z in range(5):
            off = pl.multiple_of(s * OWN + z * 128, 128)
            pltpu.sync_copy(buf, acc.at[pl.ds(off, 128)])
        pltpu.sync_copy(ones_h, buf)
        plsc.subcore_barrier()

        def dbody(j, carry):
            pltpu.sync_copy(buf.at[pl.ds(0, CHUNK)], acc.at[dstv.at[j]], add=True)
            return carry

        lax.fori_loop(0, NCHUNK, dbody, 0)
        plsc.subcore_barrier()

        for z in range(5):
            off = pl.multiple_of(s * OWN + z * 128, 128)
            pltpu.sync_copy(acc.at[pl.ds(off, 128)], buf)
            pltpu.sync_copy(buf, deg_out.at[c, s, z])

    return k(dst3, zeros128, ones128)


def _sc_msum(h, src3, dst3, zeros128):
    @functools.partial(
        pl.kernel,
        out_type=jax.ShapeDtypeStruct((2, 16, 5, 128, HID), _F32),
        mesh=_sc_mesh(),
        scratch_types=[
            pltpu.VMEM((NCHUNK, CHUNK), jnp.int32),
            pltpu.VMEM((NCHUNK, CHUNK), jnp.int32),
            pltpu.VMEM((128, HID), _F32),
            pltpu.VMEM_SHARED((NPAD, HID), _F32),
            pltpu.SemaphoreType.DMA,
        ],
    )
    def k(h_h, src_h, dst_h, z_h, out_h, srcv, dstv, gbuf, acc, sem):
        c = lax.axis_index("c")
        s = lax.axis_index("s")
        pltpu.sync_copy(src_h.at[c, s], srcv)
        pltpu.sync_copy(dst_h.at[c, s], dstv)
        pltpu.sync_copy(z_h, gbuf)
        for z in range(5):
            off = pl.multiple_of(s * OWN + z * 128, 128)
            pltpu.sync_copy(gbuf, acc.at[pl.ds(off, 128)])
        plsc.subcore_barrier()

        def body(j, carry):
            pltpu.async_copy(h_h.at[srcv.at[j]], gbuf.at[pl.ds(0, CHUNK)], sem).wait()
            pltpu.sync_copy(gbuf.at[pl.ds(0, CHUNK)], acc.at[dstv.at[j]], add=True)
            return carry

        lax.fori_loop(0, NCHUNK, body, 0)
        plsc.subcore_barrier()

        for z in range(5):
            off = pl.multiple_of(s * OWN + z * 128, 128)
            pltpu.sync_copy(acc.at[pl.ds(off, 128)], gbuf)
            pltpu.sync_copy(gbuf, out_h.at[c, s, z])

    return k(h, src3, dst3, zeros128)


def _tc_mlp(h, W_in, b_in, s_in, bb_in):
    def body(h_ref, w_ref, b_ref, s_ref, bb_ref, o_ref):
        x = h_ref[...]
        for i in range(2):
            x = jnp.dot(x, w_ref[i], preferred_element_type=_F32) + b_ref[i]
            x = jax.nn.gelu(x)
            x = _ln(x, s_ref[i], bb_ref[i])
        o_ref[...] = x

    blk = 2000
    return pl.pallas_call(
        body,
        grid=(N // blk,),
        in_specs=[
            pl.BlockSpec((blk, HID), lambda i: (i, 0)),
            pl.BlockSpec((2, HID, HID), lambda i: (0, 0, 0)),
            pl.BlockSpec((2, HID), lambda i: (0, 0)),
            pl.BlockSpec((2, HID), lambda i: (0, 0)),
            pl.BlockSpec((2, HID), lambda i: (0, 0)),
        ],
        out_specs=pl.BlockSpec((blk, HID), lambda i: (i, 0)),
        out_shape=jax.ShapeDtypeStruct((N, HID), _F32),
    )(h, W_in, b_in, s_in, bb_in)


def _tc_conv(h, msum, deg, Ws, Wn, bc, lns, lnb, Wro2, bro):
    final = Wro2 is not None
    blk = 2000

    def body(h_ref, m_ref, d_ref, ws_ref, wn_ref, bc_ref, s_ref, b_ref, *rest):
        if final:
            wro_ref, bro_ref, o_ref = rest
        else:
            (o_ref,) = rest
        x = h_ref[...]
        hn = []
        for r in range(2):
            dg = jnp.maximum(d_ref[r][:, 0:1], 1.0)
            mean = m_ref[r] / dg
            hr = (jnp.dot(x, ws_ref[r], preferred_element_type=_F32)
                  + jnp.dot(mean, wn_ref[r], preferred_element_type=_F32)
                  + bc_ref[r])
            hn.append(_ln(hr, s_ref[1 + r], b_ref[1 + r]))
        hagg = 0.5 * hn[0] + 0.5 * hn[1]
        hcur = jax.nn.gelu(_ln(hagg, s_ref[0], b_ref[0]))
        if final:
            o_ref[...] = (jnp.dot(x, wro_ref[0], preferred_element_type=_F32)
                          + jnp.dot(hcur, wro_ref[1], preferred_element_type=_F32)
                          + bro_ref[...])
        else:
            o_ref[...] = hcur

    in_specs = [
        pl.BlockSpec((blk, HID), lambda i: (i, 0)),
        pl.BlockSpec((2, blk, HID), lambda i: (0, i, 0)),
        pl.BlockSpec((2, blk, 16), lambda i: (0, i, 0)),
        pl.BlockSpec((2, HID, HID), lambda i: (0, 0, 0)),
        pl.BlockSpec((2, HID, HID), lambda i: (0, 0, 0)),
        pl.BlockSpec((2, HID), lambda i: (0, 0)),
        pl.BlockSpec((3, HID), lambda i: (0, 0)),
        pl.BlockSpec((3, HID), lambda i: (0, 0)),
    ]
    args = [h, msum, deg, Ws, Wn, bc, lns, lnb]
    if final:
        in_specs += [
            pl.BlockSpec((2, HID, HID), lambda i: (0, 0, 0)),
            pl.BlockSpec((HID,), lambda i: (0,)),
        ]
        args += [Wro2, bro]
    return pl.pallas_call(
        body,
        grid=(N // blk,),
        in_specs=in_specs,
        out_specs=pl.BlockSpec((blk, HID), lambda i: (i, 0)),
        out_shape=jax.ShapeDtypeStruct((N, HID), _F32),
    )(*args)


def kernel(feat_ids, edge_index_entail, edge_index_pathway, embed_table,
           W_in, b_in, ln_in_s, ln_in_b, W_self, W_neigh, b_conv,
           ln_c_s, ln_c_b, W_ro, b_ro):
    fid = jnp.concatenate(
        [feat_ids.astype(jnp.int32), jnp.zeros((NPAD - N,), jnp.int32)]
    ).reshape(2, 16, ENCHUNK, ECHUNK)
    src3 = jnp.stack(
        [edge_index_entail[0], edge_index_pathway[0]]
    ).astype(jnp.int32).reshape(2, 16, NCHUNK, CHUNK)
    dst3 = jnp.stack(
        [edge_index_entail[1], edge_index_pathway[1]]
    ).astype(jnp.int32).reshape(2, 16, NCHUNK, CHUNK)
    z128 = jnp.zeros((128, HID), _F32)
    ones128 = jnp.ones((128, HID), _F32)

    h0 = _sc_embed(fid, embed_table).reshape(NPAD, HID)[:N]
    deg = _sc_deg(dst3, z128, ones128).reshape(2, NPAD, HID)[:, :N, :16]
    h = _tc_mlp(h0, W_in, b_in, ln_in_s, ln_in_b)

    Wro2 = W_ro.reshape(2, HID, HID)
    for l in range(2):
        msum = _sc_msum(h, src3, dst3, z128).reshape(2, NPAD, HID)[:, :N]
        last = l == 1
        h = _tc_conv(h, msum, deg, W_self[l], W_neigh[l], b_conv[l],
                     ln_c_s[l * 3:l * 3 + 3], ln_c_b[l * 3:l * 3 + 3],
                     Wro2 if last else None, b_ro if last else None)
    return h


# speedup vs baseline: 7.2984x; 1.3304x over previous
import functools

import jax
import jax.numpy as jnp
from jax import lax
from jax.experimental import pallas as pl
from jax.experimental.pallas import tpu as pltpu
from jax.experimental.pallas import tpu_sc as plsc

N = 10000
E = 160000
HID = 128
NPAD = 10240
OWN = 640
CHUNK = 125
NCHUNK = 80
EROWS = 320
ECHUNK = 80
ENCHUNK = 4

_F32 = jnp.float32


@functools.lru_cache(maxsize=None)
def _sc_mesh():
    return plsc.VectorSubcoreMesh(
        core_axis_name="c", subcore_axis_name="s", num_cores=2, num_subcores=16
    )


def _ln(x, s, b):
    m = jnp.mean(x, axis=-1, keepdims=True)
    v = jnp.mean((x - m) ** 2, axis=-1, keepdims=True)
    return (x - m) / jnp.sqrt(v + 1e-5) * s + b


def _sc_embed(fid4, table):
    @functools.partial(
        pl.kernel,
        out_type=jax.ShapeDtypeStruct((2, 16, ENCHUNK, ECHUNK, HID), _F32),
        mesh=_sc_mesh(),
        scratch_types=[
            pltpu.VMEM((ENCHUNK, ECHUNK), jnp.int32),
            pltpu.VMEM((ECHUNK, HID), _F32),
            pltpu.SemaphoreType.DMA,
        ],
    )
    def k(fid_h, table_h, h0_out, fidv, ebuf, sem):
        c = lax.axis_index("c")
        s = lax.axis_index("s")
        pltpu.sync_copy(fid_h.at[c, s], fidv)

        def ebody(j, carry):
            pltpu.async_copy(table_h.at[fidv.at[j]], ebuf, sem).wait()
            pltpu.sync_copy(ebuf, h0_out.at[c, s, j])
            return carry

        lax.fori_loop(0, ENCHUNK, ebody, 0)

    return k(fid4, table)


def _sc_deg(dst3, zeros128, ones128):
    @functools.partial(
        pl.kernel,
        out_type=jax.ShapeDtypeStruct((2, 16, 5, 128, HID), _F32),
        mesh=_sc_mesh(),
        scratch_types=[
            pltpu.VMEM((NCHUNK, CHUNK), jnp.int32),
            pltpu.VMEM((128, HID), _F32),
            pltpu.VMEM_SHARED((NPAD, HID), _F32),
        ],
    )
    def k(dst_h, z_h, ones_h, deg_out, dstv, buf, acc):
        c = lax.axis_index("c")
        s = lax.axis_index("s")
        pltpu.sync_copy(dst_h.at[c, s], dstv)
        pltpu.sync_copy(z_h, buf)
        for z in range(5):
            off = pl.multiple_of(s * OWN + z * 128, 128)
            pltpu.sync_copy(buf, acc.at[pl.ds(off, 128)])
        pltpu.sync_copy(ones_h, buf)
        plsc.subcore_barrier()

        def dbody(j, carry):
            pltpu.sync_copy(buf.at[pl.ds(0, CHUNK)], acc.at[dstv.at[j]], add=True)
            return carry

        lax.fori_loop(0, NCHUNK, dbody, 0)
        plsc.subcore_barrier()

        for z in range(5):
            off = pl.multiple_of(s * OWN + z * 128, 128)
            pltpu.sync_copy(acc.at[pl.ds(off, 128)], buf)
            pltpu.sync_copy(buf, deg_out.at[c, s, z])

    return k(dst3, zeros128, ones128)


def _sc_msum(h, src3, dst3, zeros128):
    @functools.partial(
        pl.kernel,
        out_type=jax.ShapeDtypeStruct((2, 16, 5, 128, HID), _F32),
        mesh=_sc_mesh(),
        scratch_types=[
            pltpu.VMEM((NCHUNK // 2, CHUNK), jnp.int32),
            pltpu.VMEM((NCHUNK // 2, CHUNK), jnp.int32),
            pltpu.VMEM((128, HID), _F32),
            pltpu.VMEM((128, HID), _F32),
            pltpu.VMEM_SHARED((NPAD, HID), _F32),
            pltpu.SemaphoreType.DMA,
            pltpu.SemaphoreType.DMA,
        ],
    )
    def k(h_h, src_h, dst_h, z_h, out_h, srcv, dstv, gbuf0, gbuf1, acc,
          sem0, sem1):
        c = lax.axis_index("c")
        s = lax.axis_index("s")
        pltpu.sync_copy(z_h, gbuf0)
        for z in range(5):
            off = pl.multiple_of(s * OWN + z * 128, 128)
            pltpu.sync_copy(gbuf0, acc.at[pl.ds(off, 128)])
        plsc.subcore_barrier()

        def start(j, buf, sem):
            pltpu.async_copy(h_h.at[srcv.at[j]], buf.at[pl.ds(0, CHUNK)], sem)

        def wait(buf, sem):
            pltpu.make_async_copy(
                h_h.at[srcv.at[0]], buf.at[pl.ds(0, CHUNK)], sem).wait()

        def scat(j, buf):
            pltpu.sync_copy(buf.at[pl.ds(0, CHUNK)], acc.at[dstv.at[j]], add=True)

        half_n = NCHUNK // 2
        for half in range(2):
            pltpu.sync_copy(src_h.at[c, s, pl.ds(half * half_n, half_n)], srcv)
            pltpu.sync_copy(dst_h.at[c, s, pl.ds(half * half_n, half_n)], dstv)
            start(0, gbuf0, sem0)

            def body(i, carry):
                start(2 * i + 1, gbuf1, sem1)
                wait(gbuf0, sem0)
                scat(2 * i, gbuf0)

                @pl.when(i < half_n // 2 - 1)
                def _():
                    start(2 * i + 2, gbuf0, sem0)

                wait(gbuf1, sem1)
                scat(2 * i + 1, gbuf1)
                return carry

            lax.fori_loop(0, half_n // 2, body, 0)
        plsc.subcore_barrier()

        for z in range(5):
            off = pl.multiple_of(s * OWN + z * 128, 128)
            pltpu.sync_copy(acc.at[pl.ds(off, 128)], gbuf0)
            pltpu.sync_copy(gbuf0, out_h.at[c, s, z])

    return k(h, src3, dst3, zeros128)


def _tc_mlp(h, W_in, b_in, s_in, bb_in):
    def body(h_ref, w_ref, b_ref, s_ref, bb_ref, o_ref):
        x = h_ref[...]
        for i in range(2):
            x = jnp.dot(x, w_ref[i], preferred_element_type=_F32) + b_ref[i]
            x = jax.nn.gelu(x)
            x = _ln(x, s_ref[i], bb_ref[i])
        o_ref[...] = x

    blk = 2000
    return pl.pallas_call(
        body,
        grid=(N // blk,),
        in_specs=[
            pl.BlockSpec((blk, HID), lambda i: (i, 0)),
            pl.BlockSpec((2, HID, HID), lambda i: (0, 0, 0)),
            pl.BlockSpec((2, HID), lambda i: (0, 0)),
            pl.BlockSpec((2, HID), lambda i: (0, 0)),
            pl.BlockSpec((2, HID), lambda i: (0, 0)),
        ],
        out_specs=pl.BlockSpec((blk, HID), lambda i: (i, 0)),
        out_shape=jax.ShapeDtypeStruct((N, HID), _F32),
    )(h, W_in, b_in, s_in, bb_in)


def _tc_conv(h, msum, deg, Ws, Wn, bc, lns, lnb, Wro2, bro):
    final = Wro2 is not None
    blk = 2000

    def body(h_ref, m_ref, d_ref, ws_ref, wn_ref, bc_ref, s_ref, b_ref, *rest):
        if final:
            wro_ref, bro_ref, o_ref = rest
        else:
            (o_ref,) = rest
        x = h_ref[...]
        hn = []
        for r in range(2):
            dg = jnp.maximum(d_ref[r][:, 0:1], 1.0)
            mean = m_ref[r] / dg
            hr = (jnp.dot(x, ws_ref[r], preferred_element_type=_F32)
                  + jnp.dot(mean, wn_ref[r], preferred_element_type=_F32)
                  + bc_ref[r])
            hn.append(_ln(hr, s_ref[1 + r], b_ref[1 + r]))
        hagg = 0.5 * hn[0] + 0.5 * hn[1]
        hcur = jax.nn.gelu(_ln(hagg, s_ref[0], b_ref[0]))
        if final:
            o_ref[...] = (jnp.dot(x, wro_ref[0], preferred_element_type=_F32)
                          + jnp.dot(hcur, wro_ref[1], preferred_element_type=_F32)
                          + bro_ref[...])
        else:
            o_ref[...] = hcur

    in_specs = [
        pl.BlockSpec((blk, HID), lambda i: (i, 0)),
        pl.BlockSpec((2, blk, HID), lambda i: (0, i, 0)),
        pl.BlockSpec((2, blk, 16), lambda i: (0, i, 0)),
        pl.BlockSpec((2, HID, HID), lambda i: (0, 0, 0)),
        pl.BlockSpec((2, HID, HID), lambda i: (0, 0, 0)),
        pl.BlockSpec((2, HID), lambda i: (0, 0)),
        pl.BlockSpec((3, HID), lambda i: (0, 0)),
        pl.BlockSpec((3, HID), lambda i: (0, 0)),
    ]
    args = [h, msum, deg, Ws, Wn, bc, lns, lnb]
    if final:
        in_specs += [
            pl.BlockSpec((2, HID, HID), lambda i: (0, 0, 0)),
            pl.BlockSpec((HID,), lambda i: (0,)),
        ]
        args += [Wro2, bro]
    return pl.pallas_call(
        body,
        grid=(N // blk,),
        in_specs=in_specs,
        out_specs=pl.BlockSpec((blk, HID), lambda i: (i, 0)),
        out_shape=jax.ShapeDtypeStruct((N, HID), _F32),
    )(*args)


def kernel(feat_ids, edge_index_entail, edge_index_pathway, embed_table,
           W_in, b_in, ln_in_s, ln_in_b, W_self, W_neigh, b_conv,
           ln_c_s, ln_c_b, W_ro, b_ro):
    fid = jnp.concatenate(
        [feat_ids.astype(jnp.int32), jnp.zeros((NPAD - N,), jnp.int32)]
    ).reshape(2, 16, ENCHUNK, ECHUNK)
    src3 = jnp.stack(
        [edge_index_entail[0], edge_index_pathway[0]]
    ).astype(jnp.int32).reshape(2, 16, NCHUNK, CHUNK)
    dst3 = jnp.stack(
        [edge_index_entail[1], edge_index_pathway[1]]
    ).astype(jnp.int32).reshape(2, 16, NCHUNK, CHUNK)
    z128 = jnp.zeros((128, HID), _F32)
    ones128 = jnp.ones((128, HID), _F32)

    h0 = _sc_embed(fid, embed_table).reshape(NPAD, HID)[:N]
    deg = _sc_deg(dst3, z128, ones128).reshape(2, NPAD, HID)[:, :N, :16]
    h = _tc_mlp(h0, W_in, b_in, ln_in_s, ln_in_b)

    Wro2 = W_ro.reshape(2, HID, HID)
    for l in range(2):
        msum = _sc_msum(h, src3, dst3, z128).reshape(2, NPAD, HID)[:, :N]
        last = l == 1
        h = _tc_conv(h, msum, deg, W_self[l], W_neigh[l], b_conv[l],
                     ln_c_s[l * 3:l * 3 + 3], ln_c_b[l * 3:l * 3 + 3],
                     Wro2 if last else None, b_ro if last else None)
    return h


# speedup vs baseline: 8.4300x; 1.1550x over previous
import functools

import jax
import jax.numpy as jnp
from jax import lax
from jax.experimental import pallas as pl
from jax.experimental.pallas import tpu as pltpu
from jax.experimental.pallas import tpu_sc as plsc

N = 10000
E = 160000
HID = 128
NPAD = 10240
OWN = 640
CHUNK = 125
NCHUNK = 80
EROWS = 320
ECHUNK = 80
ENCHUNK = 4

_F32 = jnp.float32


@functools.lru_cache(maxsize=None)
def _sc_mesh():
    return plsc.VectorSubcoreMesh(
        core_axis_name="c", subcore_axis_name="s", num_cores=2, num_subcores=16
    )


def _ln(x, s, b):
    m = jnp.mean(x, axis=-1, keepdims=True)
    v = jnp.mean((x - m) ** 2, axis=-1, keepdims=True)
    return (x - m) / jnp.sqrt(v + 1e-5) * s + b


def _sc_embed(fid4, table):
    @functools.partial(
        pl.kernel,
        out_type=jax.ShapeDtypeStruct((2, 16, ENCHUNK, ECHUNK, HID), _F32),
        mesh=_sc_mesh(),
        scratch_types=[
            pltpu.VMEM((ENCHUNK, ECHUNK), jnp.int32),
            pltpu.VMEM((ECHUNK, HID), _F32),
            pltpu.VMEM((ECHUNK, HID), _F32),
            pltpu.SemaphoreType.DMA,
            pltpu.SemaphoreType.DMA,
        ],
    )
    def k(fid_h, table_h, h0_out, fidv, ebuf0, ebuf1, sem0, sem1):
        c = lax.axis_index("c")
        s = lax.axis_index("s")
        pltpu.sync_copy(fid_h.at[c, s], fidv)

        bufs = [(ebuf0, sem0), (ebuf1, sem1)]
        pltpu.async_copy(table_h.at[fidv.at[0]], ebuf0, sem0)
        for j in range(ENCHUNK):
            buf, sem = bufs[j % 2]
            if j + 1 < ENCHUNK:
                nbuf, nsem = bufs[(j + 1) % 2]
                pltpu.async_copy(table_h.at[fidv.at[j + 1]], nbuf, nsem)
            pltpu.make_async_copy(table_h.at[fidv.at[0]], buf, sem).wait()
            pltpu.sync_copy(buf, h0_out.at[c, s, j])

    return k(fid4, table)


def _sc_deg(dst3, zeros16, ones16):
    @functools.partial(
        pl.kernel,
        out_type=jax.ShapeDtypeStruct((2, 16, OWN, 16), _F32),
        mesh=_sc_mesh(),
        scratch_types=[
            pltpu.VMEM((NCHUNK, CHUNK), jnp.int32),
            pltpu.VMEM((OWN, 16), _F32),
            pltpu.VMEM_SHARED((NPAD, 16), _F32),
        ],
        compiler_params=pltpu.CompilerParams(use_tc_tiling_on_sc=False),
    )
    def k(dst_h, z_h, ones_h, deg_out, dstv, buf, acc):
        c = lax.axis_index("c")
        s = lax.axis_index("s")
        pltpu.sync_copy(dst_h.at[c, s], dstv)
        pltpu.sync_copy(z_h, buf)
        off = pl.multiple_of(s * OWN, 128)
        pltpu.sync_copy(buf, acc.at[pl.ds(off, OWN)])
        pltpu.sync_copy(ones_h, buf.at[pl.ds(0, CHUNK)])
        plsc.subcore_barrier()

        def dbody(j, carry):
            pltpu.sync_copy(buf.at[pl.ds(0, CHUNK)], acc.at[dstv.at[j]], add=True)
            return carry

        lax.fori_loop(0, NCHUNK, dbody, 0)
        plsc.subcore_barrier()

        pltpu.sync_copy(acc.at[pl.ds(off, OWN)], buf)
        pltpu.sync_copy(buf, deg_out.at[c, s])

    return k(dst3, zeros16, ones16)


def _sc_msum(h, src3, dst3, zeros128):
    @functools.partial(
        pl.kernel,
        out_type=jax.ShapeDtypeStruct((2, 16, 5, 128, HID), _F32),
        mesh=_sc_mesh(),
        scratch_types=[
            pltpu.VMEM((NCHUNK // 2, CHUNK), jnp.int32),
            pltpu.VMEM((NCHUNK // 2, CHUNK), jnp.int32),
            pltpu.VMEM((128, HID), _F32),
            pltpu.VMEM((128, HID), _F32),
            pltpu.VMEM_SHARED((NPAD, HID), _F32),
            pltpu.SemaphoreType.DMA,
            pltpu.SemaphoreType.DMA,
        ],
    )
    def k(h_h, src_h, dst_h, z_h, out_h, srcv, dstv, gbuf0, gbuf1, acc,
          sem0, sem1):
        c = lax.axis_index("c")
        s = lax.axis_index("s")
        pltpu.sync_copy(z_h, gbuf0)
        for z in range(5):
            off = pl.multiple_of(s * OWN + z * 128, 128)
            pltpu.sync_copy(gbuf0, acc.at[pl.ds(off, 128)])
        plsc.subcore_barrier()

        def start(j, buf, sem):
            pltpu.async_copy(h_h.at[srcv.at[j]], buf.at[pl.ds(0, CHUNK)], sem)

        def wait(buf, sem):
            pltpu.make_async_copy(
                h_h.at[srcv.at[0]], buf.at[pl.ds(0, CHUNK)], sem).wait()

        def scat(j, buf):
            pltpu.sync_copy(buf.at[pl.ds(0, CHUNK)], acc.at[dstv.at[j]], add=True)

        half_n = NCHUNK // 2
        for half in range(2):
            pltpu.sync_copy(src_h.at[c, s, pl.ds(half * half_n, half_n)], srcv)
            pltpu.sync_copy(dst_h.at[c, s, pl.ds(half * half_n, half_n)], dstv)
            start(0, gbuf0, sem0)

            def body(i, carry):
                start(2 * i + 1, gbuf1, sem1)
                wait(gbuf0, sem0)
                scat(2 * i, gbuf0)

                @pl.when(i < half_n // 2 - 1)
                def _():
                    start(2 * i + 2, gbuf0, sem0)

                wait(gbuf1, sem1)
                scat(2 * i + 1, gbuf1)
                return carry

            lax.fori_loop(0, half_n // 2, body, 0)
        plsc.subcore_barrier()

        for z in range(5):
            off = pl.multiple_of(s * OWN + z * 128, 128)
            pltpu.sync_copy(acc.at[pl.ds(off, 128)], gbuf0)
            pltpu.sync_copy(gbuf0, out_h.at[c, s, z])

    return k(h, src3, dst3, zeros128)


def _tc_mlp(h, W_in, b_in, s_in, bb_in):
    def body(h_ref, w_ref, b_ref, s_ref, bb_ref, o_ref):
        x = h_ref[...]
        for i in range(2):
            x = jnp.dot(x, w_ref[i], preferred_element_type=_F32) + b_ref[i]
            x = jax.nn.gelu(x)
            x = _ln(x, s_ref[i], bb_ref[i])
        o_ref[...] = x

    blk = 2000
    return pl.pallas_call(
        body,
        grid=(N // blk,),
        in_specs=[
            pl.BlockSpec((blk, HID), lambda i: (i, 0)),
            pl.BlockSpec((2, HID, HID), lambda i: (0, 0, 0)),
            pl.BlockSpec((2, HID), lambda i: (0, 0)),
            pl.BlockSpec((2, HID), lambda i: (0, 0)),
            pl.BlockSpec((2, HID), lambda i: (0, 0)),
        ],
        out_specs=pl.BlockSpec((blk, HID), lambda i: (i, 0)),
        out_shape=jax.ShapeDtypeStruct((N, HID), _F32),
    )(h, W_in, b_in, s_in, bb_in)


def _tc_conv(h, msum, deg, Ws, Wn, bc, lns, lnb, Wro2, bro):
    final = Wro2 is not None
    blk = 2000

    def body(h_ref, m_ref, d_ref, ws_ref, wn_ref, bc_ref, s_ref, b_ref, *rest):
        if final:
            wro_ref, bro_ref, o_ref = rest
        else:
            (o_ref,) = rest
        x = h_ref[...]
        hn = []
        for r in range(2):
            dg = jnp.maximum(d_ref[r][:, 0:1], 1.0)
            mean = m_ref[r] / dg
            hr = (jnp.dot(x, ws_ref[r], preferred_element_type=_F32)
                  + jnp.dot(mean, wn_ref[r], preferred_element_type=_F32)
                  + bc_ref[r])
            hn.append(_ln(hr, s_ref[1 + r], b_ref[1 + r]))
        hagg = 0.5 * hn[0] + 0.5 * hn[1]
        hcur = jax.nn.gelu(_ln(hagg, s_ref[0], b_ref[0]))
        if final:
            o_ref[...] = (jnp.dot(x, wro_ref[0], preferred_element_type=_F32)
                          + jnp.dot(hcur, wro_ref[1], preferred_element_type=_F32)
                          + bro_ref[...])
        else:
            o_ref[...] = hcur

    in_specs = [
        pl.BlockSpec((blk, HID), lambda i: (i, 0)),
        pl.BlockSpec((2, blk, HID), lambda i: (0, i, 0)),
        pl.BlockSpec((2, blk, 16), lambda i: (0, i, 0)),
        pl.BlockSpec((2, HID, HID), lambda i: (0, 0, 0)),
        pl.BlockSpec((2, HID, HID), lambda i: (0, 0, 0)),
        pl.BlockSpec((2, HID), lambda i: (0, 0)),
        pl.BlockSpec((3, HID), lambda i: (0, 0)),
        pl.BlockSpec((3, HID), lambda i: (0, 0)),
    ]
    args = [h, msum, deg, Ws, Wn, bc, lns, lnb]
    if final:
        in_specs += [
            pl.BlockSpec((2, HID, HID), lambda i: (0, 0, 0)),
            pl.BlockSpec((HID,), lambda i: (0,)),
        ]
        args += [Wro2, bro]
    return pl.pallas_call(
        body,
        grid=(N // blk,),
        in_specs=in_specs,
        out_specs=pl.BlockSpec((blk, HID), lambda i: (i, 0)),
        out_shape=jax.ShapeDtypeStruct((N, HID), _F32),
    )(*args)


def kernel(feat_ids, edge_index_entail, edge_index_pathway, embed_table,
           W_in, b_in, ln_in_s, ln_in_b, W_self, W_neigh, b_conv,
           ln_c_s, ln_c_b, W_ro, b_ro):
    fid = jnp.concatenate(
        [feat_ids.astype(jnp.int32), jnp.zeros((NPAD - N,), jnp.int32)]
    ).reshape(2, 16, ENCHUNK, ECHUNK)
    src3 = jnp.stack(
        [edge_index_entail[0], edge_index_pathway[0]]
    ).astype(jnp.int32).reshape(2, 16, NCHUNK, CHUNK)
    dst3 = jnp.stack(
        [edge_index_entail[1], edge_index_pathway[1]]
    ).astype(jnp.int32).reshape(2, 16, NCHUNK, CHUNK)
    z128 = jnp.zeros((128, HID), _F32)
    z16 = jnp.zeros((OWN, 16), _F32)
    ones16 = jnp.ones((CHUNK, 16), _F32)

    h0 = _sc_embed(fid, embed_table).reshape(NPAD, HID)[:N]
    deg = _sc_deg(dst3, z16, ones16).reshape(2, NPAD, 16)[:, :N]
    h = _tc_mlp(h0, W_in, b_in, ln_in_s, ln_in_b)

    Wro2 = W_ro.reshape(2, HID, HID)
    for l in range(2):
        msum = _sc_msum(h, src3, dst3, z128).reshape(2, NPAD, HID)[:, :N]
        last = l == 1
        h = _tc_conv(h, msum, deg, W_self[l], W_neigh[l], b_conv[l],
                     ln_c_s[l * 3:l * 3 + 3], ln_c_b[l * 3:l * 3 + 3],
                     Wro2 if last else None, b_ro if last else None)
    return h


# speedup vs baseline: 8.8006x; 1.0440x over previous
import functools

import jax
import jax.numpy as jnp
from jax import lax
from jax.experimental import pallas as pl
from jax.experimental.pallas import tpu as pltpu
from jax.experimental.pallas import tpu_sc as plsc

N = 10000
E = 160000
HID = 128
NPAD = 10240
OWN = 640
CHUNK = 125
NCHUNK = 80
EROWS = 320
ECHUNK = 80
ENCHUNK = 4

_F32 = jnp.float32


@functools.lru_cache(maxsize=None)
def _sc_mesh():
    return plsc.VectorSubcoreMesh(
        core_axis_name="c", subcore_axis_name="s", num_cores=2, num_subcores=16
    )


def _ln(x, s, b):
    m = jnp.mean(x, axis=-1, keepdims=True)
    v = jnp.mean((x - m) ** 2, axis=-1, keepdims=True)
    return (x - m) / jnp.sqrt(v + 1e-5) * s + b


def _sc_embed(fid4, table):
    @functools.partial(
        pl.kernel,
        out_type=jax.ShapeDtypeStruct((2, 16, ENCHUNK, ECHUNK, HID), _F32),
        mesh=_sc_mesh(),
        scratch_types=[
            pltpu.VMEM((ENCHUNK, ECHUNK), jnp.int32),
            pltpu.VMEM((ECHUNK, HID), _F32),
            pltpu.VMEM((ECHUNK, HID), _F32),
            pltpu.SemaphoreType.DMA,
            pltpu.SemaphoreType.DMA,
        ],
    )
    def k(fid_h, table_h, h0_out, fidv, ebuf0, ebuf1, sem0, sem1):
        c = lax.axis_index("c")
        s = lax.axis_index("s")
        pltpu.sync_copy(fid_h.at[c, s], fidv)

        bufs = [(ebuf0, sem0), (ebuf1, sem1)]
        pltpu.async_copy(table_h.at[fidv.at[0]], ebuf0, sem0)
        for j in range(ENCHUNK):
            buf, sem = bufs[j % 2]
            if j + 1 < ENCHUNK:
                nbuf, nsem = bufs[(j + 1) % 2]
                pltpu.async_copy(table_h.at[fidv.at[j + 1]], nbuf, nsem)
            pltpu.make_async_copy(table_h.at[fidv.at[0]], buf, sem).wait()
            pltpu.sync_copy(buf, h0_out.at[c, s, j])

    return k(fid4, table)


def _sc_deg(dst3, zeros16, ones16):
    @functools.partial(
        pl.kernel,
        out_type=jax.ShapeDtypeStruct((2, 16, OWN, 16), _F32),
        mesh=_sc_mesh(),
        scratch_types=[
            pltpu.VMEM((NCHUNK, CHUNK), jnp.int32),
            pltpu.VMEM((OWN, 16), _F32),
            pltpu.VMEM_SHARED((NPAD, 16), _F32),
        ],
        compiler_params=pltpu.CompilerParams(use_tc_tiling_on_sc=False),
    )
    def k(dst_h, z_h, ones_h, deg_out, dstv, buf, acc):
        c = lax.axis_index("c")
        s = lax.axis_index("s")
        pltpu.sync_copy(dst_h.at[c, s], dstv)
        pltpu.sync_copy(z_h, buf)
        off = pl.multiple_of(s * OWN, 128)
        pltpu.sync_copy(buf, acc.at[pl.ds(off, OWN)])
        pltpu.sync_copy(ones_h, buf.at[pl.ds(0, CHUNK)])
        plsc.subcore_barrier()

        def dbody(j, carry):
            pltpu.sync_copy(buf.at[pl.ds(0, CHUNK)], acc.at[dstv.at[j]], add=True)
            return carry

        lax.fori_loop(0, NCHUNK, dbody, 0)
        plsc.subcore_barrier()

        pltpu.sync_copy(acc.at[pl.ds(off, OWN)], buf)
        pltpu.sync_copy(buf, deg_out.at[c, s])

    return k(dst3, zeros16, ones16)


def _sc_msum(h, src3, dst3, zeros128):
    @functools.partial(
        pl.kernel,
        out_type=jax.ShapeDtypeStruct((2, 16, OWN, HID), _F32),
        mesh=_sc_mesh(),
        scratch_types=[
            pltpu.VMEM((NCHUNK // 2, CHUNK), jnp.int32),
            pltpu.VMEM((NCHUNK // 2, CHUNK), jnp.int32),
            pltpu.VMEM((128, HID), _F32),
            pltpu.VMEM((128, HID), _F32),
            pltpu.VMEM_SHARED((NPAD, HID), _F32),
            pltpu.SemaphoreType.DMA,
            pltpu.SemaphoreType.DMA,
        ],
    )
    def k(h_h, src_h, dst_h, z_h, out_h, srcv, dstv, gbuf0, gbuf1, acc,
          sem0, sem1):
        c = lax.axis_index("c")
        s = lax.axis_index("s")
        pltpu.sync_copy(z_h, gbuf0)
        for z in range(5):
            off = pl.multiple_of(s * OWN + z * 128, 128)
            pltpu.sync_copy(gbuf0, acc.at[pl.ds(off, 128)])
        plsc.subcore_barrier()

        def start(j, buf, sem):
            pltpu.async_copy(h_h.at[srcv.at[j]], buf.at[pl.ds(0, CHUNK)], sem)

        def wait(buf, sem):
            pltpu.make_async_copy(
                h_h.at[srcv.at[0]], buf.at[pl.ds(0, CHUNK)], sem).wait()

        def scat(j, buf):
            pltpu.sync_copy(buf.at[pl.ds(0, CHUNK)], acc.at[dstv.at[j]], add=True)

        half_n = NCHUNK // 2
        for half in range(2):
            pltpu.sync_copy(src_h.at[c, s, pl.ds(half * half_n, half_n)], srcv)
            pltpu.sync_copy(dst_h.at[c, s, pl.ds(half * half_n, half_n)], dstv)
            start(0, gbuf0, sem0)

            def body(i, carry):
                start(2 * i + 1, gbuf1, sem1)
                wait(gbuf0, sem0)
                scat(2 * i, gbuf0)

                @pl.when(i < half_n // 2 - 1)
                def _():
                    start(2 * i + 2, gbuf0, sem0)

                wait(gbuf1, sem1)
                scat(2 * i + 1, gbuf1)
                return carry

            lax.fori_loop(0, half_n // 2, body, 0)
        plsc.subcore_barrier()

        pltpu.sync_copy(acc.at[pl.ds(pl.multiple_of(s * OWN, 128), OWN)],
                        out_h.at[c, s])

    return k(h, src3, dst3, zeros128)


def _tc_mlp(h, W_in, b_in, s_in, bb_in):
    def body(h_ref, w_ref, b_ref, s_ref, bb_ref, o_ref):
        x = h_ref[...]
        for i in range(2):
            x = jnp.dot(x, w_ref[i], preferred_element_type=_F32) + b_ref[i]
            x = jax.nn.gelu(x)
            x = _ln(x, s_ref[i], bb_ref[i])
        o_ref[...] = x

    blk = 2048
    return pl.pallas_call(
        body,
        grid=(NPAD // blk,),
        in_specs=[
            pl.BlockSpec((blk, HID), lambda i: (i, 0)),
            pl.BlockSpec((2, HID, HID), lambda i: (0, 0, 0)),
            pl.BlockSpec((2, HID), lambda i: (0, 0)),
            pl.BlockSpec((2, HID), lambda i: (0, 0)),
            pl.BlockSpec((2, HID), lambda i: (0, 0)),
        ],
        out_specs=pl.BlockSpec((blk, HID), lambda i: (i, 0)),
        out_shape=jax.ShapeDtypeStruct((NPAD, HID), _F32),
    )(h, W_in, b_in, s_in, bb_in)


def _tc_conv(h, msum, deg, Ws, Wn, bc, lns, lnb, Wro2, bro):
    final = Wro2 is not None
    blk = 2048

    def body(h_ref, m_ref, d_ref, ws_ref, wn_ref, bc_ref, s_ref, b_ref, *rest):
        if final:
            wro_ref, bro_ref, o_ref = rest
        else:
            (o_ref,) = rest
        x = h_ref[...]
        hn = []
        for r in range(2):
            dg = jnp.maximum(d_ref[r][:, 0:1], 1.0)
            mean = m_ref[r] / dg
            hr = (jnp.dot(x, ws_ref[r], preferred_element_type=_F32)
                  + jnp.dot(mean, wn_ref[r], preferred_element_type=_F32)
                  + bc_ref[r])
            hn.append(_ln(hr, s_ref[1 + r], b_ref[1 + r]))
        hagg = 0.5 * hn[0] + 0.5 * hn[1]
        hcur = jax.nn.gelu(_ln(hagg, s_ref[0], b_ref[0]))
        if final:
            o_ref[...] = (jnp.dot(x, wro_ref[0], preferred_element_type=_F32)
                          + jnp.dot(hcur, wro_ref[1], preferred_element_type=_F32)
                          + bro_ref[...])
        else:
            o_ref[...] = hcur

    in_specs = [
        pl.BlockSpec((blk, HID), lambda i: (i, 0)),
        pl.BlockSpec((2, blk, HID), lambda i: (0, i, 0)),
        pl.BlockSpec((2, blk, 16), lambda i: (0, i, 0)),
        pl.BlockSpec((2, HID, HID), lambda i: (0, 0, 0)),
        pl.BlockSpec((2, HID, HID), lambda i: (0, 0, 0)),
        pl.BlockSpec((2, HID), lambda i: (0, 0)),
        pl.BlockSpec((3, HID), lambda i: (0, 0)),
        pl.BlockSpec((3, HID), lambda i: (0, 0)),
    ]
    args = [h, msum, deg, Ws, Wn, bc, lns, lnb]
    if final:
        in_specs += [
            pl.BlockSpec((2, HID, HID), lambda i: (0, 0, 0)),
            pl.BlockSpec((HID,), lambda i: (0,)),
        ]
        args += [Wro2, bro]
    return pl.pallas_call(
        body,
        grid=(NPAD // blk,),
        in_specs=in_specs,
        out_specs=pl.BlockSpec((blk, HID), lambda i: (i, 0)),
        out_shape=jax.ShapeDtypeStruct((NPAD, HID), _F32),
    )(*args)


def kernel(feat_ids, edge_index_entail, edge_index_pathway, embed_table,
           W_in, b_in, ln_in_s, ln_in_b, W_self, W_neigh, b_conv,
           ln_c_s, ln_c_b, W_ro, b_ro):
    fid = jnp.concatenate(
        [feat_ids.astype(jnp.int32), jnp.zeros((NPAD - N,), jnp.int32)]
    ).reshape(2, 16, ENCHUNK, ECHUNK)
    src3 = jnp.stack(
        [edge_index_entail[0], edge_index_pathway[0]]
    ).astype(jnp.int32).reshape(2, 16, NCHUNK, CHUNK)
    dst3 = jnp.stack(
        [edge_index_entail[1], edge_index_pathway[1]]
    ).astype(jnp.int32).reshape(2, 16, NCHUNK, CHUNK)
    z128 = jnp.zeros((128, HID), _F32)
    z16 = jnp.zeros((OWN, 16), _F32)
    ones16 = jnp.ones((CHUNK, 16), _F32)

    h0 = _sc_embed(fid, embed_table).reshape(NPAD, HID)
    deg = _sc_deg(dst3, z16, ones16).reshape(2, NPAD, 16)
    h = _tc_mlp(h0, W_in, b_in, ln_in_s, ln_in_b)

    Wro2 = W_ro.reshape(2, HID, HID)
    for l in range(2):
        msum = _sc_msum(h, src3, dst3, z128).reshape(2, NPAD, HID)
        last = l == 1
        h = _tc_conv(h, msum, deg, W_self[l], W_neigh[l], b_conv[l],
                     ln_c_s[l * 3:l * 3 + 3], ln_c_b[l * 3:l * 3 + 3],
                     Wro2 if last else None, b_ro if last else None)
    return h[:N]


# speedup vs baseline: 8.9140x; 1.0129x over previous
import functools

import jax
import jax.numpy as jnp
from jax import lax
from jax.experimental import pallas as pl
from jax.experimental.pallas import tpu as pltpu
from jax.experimental.pallas import tpu_sc as plsc

N = 10000
E = 160000
HID = 128
NPAD = 10240
OWN = 640
CHUNK = 125
NCHUNK = 80
EROWS = 320
ECHUNK = 80
ENCHUNK = 4

_F32 = jnp.float32


@functools.lru_cache(maxsize=None)
def _sc_mesh():
    return plsc.VectorSubcoreMesh(
        core_axis_name="c", subcore_axis_name="s", num_cores=2, num_subcores=16
    )


def _ln(x, s, b):
    m = jnp.mean(x, axis=-1, keepdims=True)
    v = jnp.mean((x - m) ** 2, axis=-1, keepdims=True)
    return (x - m) / jnp.sqrt(v + 1e-5) * s + b


def _sc_embed(fid4, table):
    @functools.partial(
        pl.kernel,
        out_type=jax.ShapeDtypeStruct((2, 16, ENCHUNK, ECHUNK, HID), _F32),
        mesh=_sc_mesh(),
        scratch_types=[
            pltpu.VMEM((ENCHUNK, ECHUNK), jnp.int32),
            pltpu.VMEM((ECHUNK, HID), _F32),
            pltpu.VMEM((ECHUNK, HID), _F32),
            pltpu.SemaphoreType.DMA,
            pltpu.SemaphoreType.DMA,
        ],
    )
    def k(fid_h, table_h, h0_out, fidv, ebuf0, ebuf1, sem0, sem1):
        c = lax.axis_index("c")
        s = lax.axis_index("s")
        pltpu.sync_copy(fid_h.at[c, s], fidv)

        bufs = [(ebuf0, sem0), (ebuf1, sem1)]
        pltpu.async_copy(table_h.at[fidv.at[0]], ebuf0, sem0)
        for j in range(ENCHUNK):
            buf, sem = bufs[j % 2]
            if j + 1 < ENCHUNK:
                nbuf, nsem = bufs[(j + 1) % 2]
                pltpu.async_copy(table_h.at[fidv.at[j + 1]], nbuf, nsem)
            pltpu.make_async_copy(table_h.at[fidv.at[0]], buf, sem).wait()
            pltpu.sync_copy(buf, h0_out.at[c, s, j])

    return k(fid4, table)


def _sc_deg(dst3, zeros16, ones16):
    @functools.partial(
        pl.kernel,
        out_type=jax.ShapeDtypeStruct((2, 16, OWN, 16), _F32),
        mesh=_sc_mesh(),
        scratch_types=[
            pltpu.VMEM((NCHUNK, CHUNK), jnp.int32),
            pltpu.VMEM((OWN, 16), _F32),
            pltpu.VMEM_SHARED((NPAD, 16), _F32),
        ],
        compiler_params=pltpu.CompilerParams(use_tc_tiling_on_sc=False),
    )
    def k(dst_h, z_h, ones_h, deg_out, dstv, buf, acc):
        c = lax.axis_index("c")
        s = lax.axis_index("s")
        pltpu.sync_copy(dst_h.at[c, s], dstv)
        pltpu.sync_copy(z_h, buf)
        off = pl.multiple_of(s * OWN, 128)
        pltpu.sync_copy(buf, acc.at[pl.ds(off, OWN)])
        pltpu.sync_copy(ones_h, buf.at[pl.ds(0, CHUNK)])
        plsc.subcore_barrier()

        def dbody(j, carry):
            pltpu.sync_copy(buf.at[pl.ds(0, CHUNK)], acc.at[dstv.at[j]], add=True)
            return carry

        lax.fori_loop(0, NCHUNK, dbody, 0)
        plsc.subcore_barrier()

        pltpu.sync_copy(acc.at[pl.ds(off, OWN)], buf)
        pltpu.sync_copy(buf, deg_out.at[c, s])

    return k(dst3, zeros16, ones16)


def _sc_msum(h, src3, dst3, zeros128):
    @functools.partial(
        pl.kernel,
        out_type=jax.ShapeDtypeStruct((2, 16, OWN, HID), _F32),
        mesh=_sc_mesh(),
        scratch_types=[
            pltpu.VMEM((NCHUNK // 2, CHUNK), jnp.int32),
            pltpu.VMEM((NCHUNK // 2, CHUNK), jnp.int32),
            pltpu.VMEM((128, HID), _F32),
            pltpu.VMEM((128, HID), _F32),
            pltpu.VMEM_SHARED((NPAD, HID), _F32),
            pltpu.SemaphoreType.DMA,
            pltpu.SemaphoreType.DMA,
        ],
    )
    def k(h_h, src_h, dst_h, z_h, out_h, srcv, dstv, gbuf0, gbuf1, acc,
          sem0, sem1):
        c = lax.axis_index("c")
        s = lax.axis_index("s")
        pltpu.sync_copy(z_h, gbuf0)
        for z in range(5):
            off = pl.multiple_of(s * OWN + z * 128, 128)
            pltpu.sync_copy(gbuf0, acc.at[pl.ds(off, 128)])
        plsc.subcore_barrier()

        def start(j, buf, sem):
            pltpu.async_copy(h_h.at[srcv.at[j]], buf.at[pl.ds(0, CHUNK)], sem)

        def wait(buf, sem):
            pltpu.make_async_copy(
                h_h.at[srcv.at[0]], buf.at[pl.ds(0, CHUNK)], sem).wait()

        def scat(j, buf):
            pltpu.sync_copy(buf.at[pl.ds(0, CHUNK)], acc.at[dstv.at[j]], add=True)

        half_n = NCHUNK // 2
        for half in range(2):
            pltpu.sync_copy(src_h.at[c, s, pl.ds(half * half_n, half_n)], srcv)
            pltpu.sync_copy(dst_h.at[c, s, pl.ds(half * half_n, half_n)], dstv)
            start(0, gbuf0, sem0)

            def body(i, carry):
                start(2 * i + 1, gbuf1, sem1)
                wait(gbuf0, sem0)
                scat(2 * i, gbuf0)

                @pl.when(i < half_n // 2 - 1)
                def _():
                    start(2 * i + 2, gbuf0, sem0)

                wait(gbuf1, sem1)
                scat(2 * i + 1, gbuf1)
                return carry

            lax.fori_loop(0, half_n // 2, body, 0)
        plsc.subcore_barrier()

        pltpu.sync_copy(acc.at[pl.ds(pl.multiple_of(s * OWN, 128), OWN)],
                        out_h.at[c, s])

    return k(h, src3, dst3, zeros128)


def _tc_mlp(h, W_in, b_in, s_in, bb_in):
    def body(h_ref, w_ref, b_ref, s_ref, bb_ref, o_ref):
        x = h_ref[...]
        for i in range(2):
            x = jnp.dot(x, w_ref[i], preferred_element_type=_F32) + b_ref[i]
            x = jax.nn.gelu(x)
            x = _ln(x, s_ref[i], bb_ref[i])
        o_ref[...] = x

    blk = 2048
    return pl.pallas_call(
        body,
        grid=(NPAD // blk,),
        in_specs=[
            pl.BlockSpec((blk, HID), lambda i: (i, 0)),
            pl.BlockSpec((2, HID, HID), lambda i: (0, 0, 0)),
            pl.BlockSpec((2, HID), lambda i: (0, 0)),
            pl.BlockSpec((2, HID), lambda i: (0, 0)),
            pl.BlockSpec((2, HID), lambda i: (0, 0)),
        ],
        out_specs=pl.BlockSpec((blk, HID), lambda i: (i, 0)),
        out_shape=jax.ShapeDtypeStruct((NPAD, HID), _F32),
    )(h, W_in, b_in, s_in, bb_in)


def _tc_conv(h, msum, deg, Ws, Wn, bc, lns, lnb, Wro2, bro):
    final = Wro2 is not None
    blk = 2000 if final else 2048
    nrows = N if final else NPAD

    def body(h_ref, m_ref, d_ref, ws_ref, wn_ref, bc_ref, s_ref, b_ref, *rest):
        if final:
            wro_ref, bro_ref, o_ref = rest
        else:
            (o_ref,) = rest
        x = h_ref[...]
        hn = []
        for r in range(2):
            dg = jnp.maximum(d_ref[r][:, 0:1], 1.0)
            mean = m_ref[r] / dg
            hr = (jnp.dot(x, ws_ref[r], preferred_element_type=_F32)
                  + jnp.dot(mean, wn_ref[r], preferred_element_type=_F32)
                  + bc_ref[r])
            hn.append(_ln(hr, s_ref[1 + r], b_ref[1 + r]))
        hagg = 0.5 * hn[0] + 0.5 * hn[1]
        hcur = jax.nn.gelu(_ln(hagg, s_ref[0], b_ref[0]))
        if final:
            o_ref[...] = (jnp.dot(x, wro_ref[0], preferred_element_type=_F32)
                          + jnp.dot(hcur, wro_ref[1], preferred_element_type=_F32)
                          + bro_ref[...])
        else:
            o_ref[...] = hcur

    in_specs = [
        pl.BlockSpec((blk, HID), lambda i: (i, 0)),
        pl.BlockSpec((2, blk, HID), lambda i: (0, i, 0)),
        pl.BlockSpec((2, blk, 16), lambda i: (0, i, 0)),
        pl.BlockSpec((2, HID, HID), lambda i: (0, 0, 0)),
        pl.BlockSpec((2, HID, HID), lambda i: (0, 0, 0)),
        pl.BlockSpec((2, HID), lambda i: (0, 0)),
        pl.BlockSpec((3, HID), lambda i: (0, 0)),
        pl.BlockSpec((3, HID), lambda i: (0, 0)),
    ]
    args = [h, msum, deg, Ws, Wn, bc, lns, lnb]
    if final:
        in_specs += [
            pl.BlockSpec((2, HID, HID), lambda i: (0, 0, 0)),
            pl.BlockSpec((HID,), lambda i: (0,)),
        ]
        args += [Wro2, bro]
    return pl.pallas_call(
        body,
        grid=(nrows // blk,),
        in_specs=in_specs,
        out_specs=pl.BlockSpec((blk, HID), lambda i: (i, 0)),
        out_shape=jax.ShapeDtypeStruct((nrows, HID), _F32),
    )(*args)


def kernel(feat_ids, edge_index_entail, edge_index_pathway, embed_table,
           W_in, b_in, ln_in_s, ln_in_b, W_self, W_neigh, b_conv,
           ln_c_s, ln_c_b, W_ro, b_ro):
    fid = jnp.concatenate(
        [feat_ids.astype(jnp.int32), jnp.zeros((NPAD - N,), jnp.int32)]
    ).reshape(2, 16, ENCHUNK, ECHUNK)
    src3 = jnp.stack(
        [edge_index_entail[0], edge_index_pathway[0]]
    ).astype(jnp.int32).reshape(2, 16, NCHUNK, CHUNK)
    dst3 = jnp.stack(
        [edge_index_entail[1], edge_index_pathway[1]]
    ).astype(jnp.int32).reshape(2, 16, NCHUNK, CHUNK)
    z128 = jnp.zeros((128, HID), _F32)
    z16 = jnp.zeros((OWN, 16), _F32)
    ones16 = jnp.ones((CHUNK, 16), _F32)

    h0 = _sc_embed(fid, embed_table).reshape(NPAD, HID)
    deg = _sc_deg(dst3, z16, ones16).reshape(2, NPAD, 16)
    h = _tc_mlp(h0, W_in, b_in, ln_in_s, ln_in_b)

    Wro2 = W_ro.reshape(2, HID, HID)
    for l in range(2):
        msum = _sc_msum(h, src3, dst3, z128).reshape(2, NPAD, HID)
        last = l == 1
        h = _tc_conv(h, msum, deg, W_self[l], W_neigh[l], b_conv[l],
                     ln_c_s[l * 3:l * 3 + 3], ln_c_b[l * 3:l * 3 + 3],
                     Wro2 if last else None, b_ro if last else None)
    return h
